# Initial kernel scaffold; baseline (speedup 1.0000x reference)
import jax
import jax.numpy as jnp
from jax.experimental import pallas as pl


def kernel(img_emb, cap_emb, labels):
    raise NotImplementedError("write your pallas kernel here")


# speedup vs baseline: 24.7409x; 24.7409x over previous
import jax
import jax.numpy as jnp
from jax.experimental import pallas as pl

_N = 512
_H = 256
_T = 0.6
_EPS = 1e-6
_MARGIN = 1.0
_LAMBDA_C = 0.5


def _loss_kernel(x_ref, out_ref):
    x = x_ref[...]
    n = jnp.sqrt(jnp.sum(x * x, axis=1, keepdims=True))
    z = x / jnp.maximum(n, 1e-12)

    sim = jax.lax.dot_general(
        z, z, (((1,), (1,)), ((), ())),
        precision=jax.lax.Precision.HIGHEST,
        preferred_element_type=jnp.float32)

    i = jax.lax.broadcasted_iota(jnp.int32, (_N, _N), 0)
    j = jax.lax.broadcasted_iota(jnp.int32, (_N, _N), 1)

    same = ((i - j) % _H) == 0
    hard = jnp.where(same, 0.0, sim)

    m = jnp.max(hard, axis=1, keepdims=True)
    cand = jnp.where(hard == m, j, _N)
    idx = jnp.min(cand, axis=1, keepdims=True)
    member = jnp.any(idx == j, axis=0, keepdims=True)

    pos_sel = ((j == i + 128) & (i < 128)) | ((j == i - 128) & (i >= 128) & (i < _H))
    sum_pos = jnp.sum(jnp.where(pos_sel, sim, 0.0), keepdims=True)
    contrastive = (jnp.sum(m, keepdims=True) - 2.0 * sum_pos) / (_T * _H)

    sq = jnp.sum(z * z, axis=1, keepdims=True)
    srow = jnp.sum(z, axis=1, keepdims=True)
    d2 = ((sq[:_H] + 2.0 * _EPS * srow[:_H])
          + (sq.T - 2.0 * _EPS * srow.T)
          - 2.0 * sim[:_H, :]
          + (_N * _EPS * _EPS))
    negd2 = jnp.min(jnp.where(member, d2, jnp.float32(3.4e38)),
                    axis=1, keepdims=True)
    neg_dist = jnp.sqrt(jnp.maximum(negd2, 0.0))

    dpos = z[:_H, :] - z[_H:, :] + _EPS
    pos_dist = jnp.sqrt(jnp.sum(dpos * dpos, axis=1, keepdims=True))

    triplet = jnp.sum(jnp.maximum(pos_dist - neg_dist + _MARGIN, 0.0),
                      keepdims=True) / _H
    out_ref[...] = triplet + _LAMBDA_C * contrastive


def kernel(img_emb, cap_emb, labels):
    x = jnp.concatenate([img_emb, cap_emb], axis=0)
    out = pl.pallas_call(
        _loss_kernel,
        out_shape=jax.ShapeDtypeStruct((1, 1), jnp.float32),
    )(x.astype(jnp.float32))
    return out[0, 0]


# speedup vs baseline: 35.3290x; 1.4280x over previous
import jax
import jax.numpy as jnp
from jax.experimental import pallas as pl

_N = 512
_H = 256
_T = 0.6
_EPS = 1e-6
_MARGIN = 1.0
_LAMBDA_C = 0.5


def _loss_kernel(img_ref, cap_ref, out_ref):
    x = jnp.concatenate([img_ref[...], cap_ref[...]], axis=0)
    n = jnp.sqrt(jnp.sum(x * x, axis=1, keepdims=True))
    z = x / jnp.maximum(n, 1e-12)

    sim = jax.lax.dot_general(
        z, z, (((1,), (1,)), ((), ())),
        precision=jax.lax.Precision.HIGHEST,
        preferred_element_type=jnp.float32)

    i = jax.lax.broadcasted_iota(jnp.int32, (_N, _N), 0)
    j = jax.lax.broadcasted_iota(jnp.int32, (_N, _N), 1)

    same = ((i - j) % _H) == 0
    hard = jnp.where(same, 0.0, sim)

    m = jnp.max(hard, axis=1, keepdims=True)
    cand = jnp.where(hard == m, j, _N)
    idx = jnp.min(cand, axis=1, keepdims=True)
    member = jnp.any(idx == j, axis=0, keepdims=True)

    pos_sel = ((j == i + 128) & (i < 128)) | ((j == i - 128) & (i >= 128) & (i < _H))
    sum_pos = jnp.sum(jnp.where(pos_sel, sim, 0.0), keepdims=True)
    contrastive = (jnp.sum(m, keepdims=True) - 2.0 * sum_pos) / (_T * _H)

    sq = jnp.sum(z * z, axis=1, keepdims=True)
    srow = jnp.sum(z, axis=1, keepdims=True)
    d2 = ((sq[:_H] + 2.0 * _EPS * srow[:_H])
          + (sq.T - 2.0 * _EPS * srow.T)
          - 2.0 * sim[:_H, :]
          + (_N * _EPS * _EPS))
    negd2 = jnp.min(jnp.where(member, d2, jnp.float32(3.4e38)),
                    axis=1, keepdims=True)
    neg_dist = jnp.sqrt(jnp.maximum(negd2, 0.0))

    dpos = z[:_H, :] - z[_H:, :] + _EPS
    pos_dist = jnp.sqrt(jnp.sum(dpos * dpos, axis=1, keepdims=True))

    triplet = jnp.sum(jnp.maximum(pos_dist - neg_dist + _MARGIN, 0.0),
                      keepdims=True) / _H
    out_ref[...] = triplet + _LAMBDA_C * contrastive


def kernel(img_emb, cap_emb, labels):
    out = pl.pallas_call(
        _loss_kernel,
        out_shape=jax.ShapeDtypeStruct((1, 1), jnp.float32),
    )(img_emb.astype(jnp.float32), cap_emb.astype(jnp.float32))
    return out[0, 0]


# speedup vs baseline: 48.5518x; 1.3743x over previous
import jax
import jax.numpy as jnp
from jax.experimental import pallas as pl

_N = 512
_H = 256
_T = 0.6
_EPS = 1e-6
_MARGIN = 1.0
_LAMBDA_C = 0.5


def _loss_kernel(img_ref, cap_ref, out_ref):
    x = jnp.concatenate([img_ref[...], cap_ref[...]], axis=0)
    n = jnp.sqrt(jnp.sum(x * x, axis=1, keepdims=True))
    z = x / jnp.maximum(n, 1e-12)

    sim = jax.lax.dot_general(
        z, z, (((1,), (1,)), ((), ())),
        preferred_element_type=jnp.float32)

    i = jax.lax.broadcasted_iota(jnp.int32, (_N, _N), 0)
    j = jax.lax.broadcasted_iota(jnp.int32, (_N, _N), 1)

    same = ((i - j) % _H) == 0
    hard = jnp.where(same, 0.0, sim)

    m = jnp.max(hard, axis=1, keepdims=True)
    cand = jnp.where(hard == m, j, _N)
    idx = jnp.min(cand, axis=1, keepdims=True)
    member = jnp.any(idx == j, axis=0, keepdims=True)

    pos_sel = ((j == i + 128) & (i < 128)) | ((j == i - 128) & (i >= 128) & (i < _H))
    sum_pos = jnp.sum(jnp.where(pos_sel, sim, 0.0), keepdims=True)
    contrastive = (jnp.sum(m, keepdims=True) - 2.0 * sum_pos) / (_T * _H)

    sq = jnp.sum(z * z, axis=1, keepdims=True)
    srow = jnp.sum(z, axis=1, keepdims=True)
    d2 = ((sq[:_H] + 2.0 * _EPS * srow[:_H])
          + (sq.T - 2.0 * _EPS * srow.T)
          - 2.0 * sim[:_H, :]
          + (_N * _EPS * _EPS))
    negd2 = jnp.min(jnp.where(member, d2, jnp.float32(3.4e38)),
                    axis=1, keepdims=True)
    neg_dist = jnp.sqrt(jnp.maximum(negd2, 0.0))

    dpos = z[:_H, :] - z[_H:, :] + _EPS
    pos_dist = jnp.sqrt(jnp.sum(dpos * dpos, axis=1, keepdims=True))

    triplet = jnp.sum(jnp.maximum(pos_dist - neg_dist + _MARGIN, 0.0),
                      keepdims=True) / _H
    out_ref[...] = triplet + _LAMBDA_C * contrastive


def kernel(img_emb, cap_emb, labels):
    out = pl.pallas_call(
        _loss_kernel,
        out_shape=jax.ShapeDtypeStruct((1, 1), jnp.float32),
    )(img_emb.astype(jnp.float32), cap_emb.astype(jnp.float32))
    return out[0, 0]


# speedup vs baseline: 52.2226x; 1.0756x over previous
import jax
import jax.numpy as jnp
from jax.experimental import pallas as pl

_H = 256
_T = 0.6
_EPS = 1e-6
_MARGIN = 1.0
_LAMBDA_C = 0.5
_BIG = 3.4e38


def _dot_t(a, b):
    return jax.lax.dot_general(a, b, (((1,), (1,)), ((), ())),
                               preferred_element_type=jnp.float32)


def _loss_kernel(img_ref, cap_ref, out_ref):
    xi = img_ref[...]
    xj = cap_ref[...]
    ri = 1.0 / jnp.maximum(jnp.sqrt(jnp.sum(xi * xi, axis=1, keepdims=True)), 1e-12)
    rj = 1.0 / jnp.maximum(jnp.sqrt(jnp.sum(xj * xj, axis=1, keepdims=True)), 1e-12)
    zi = xi * ri
    zj = xj * rj

    sim_tl = _dot_t(zi, zi)
    sim_tr = _dot_t(zi, zj)
    sim_bb = _dot_t(zj, zj)

    i2 = jax.lax.broadcasted_iota(jnp.int32, (_H, _H), 0)
    j2 = jax.lax.broadcasted_iota(jnp.int32, (_H, _H), 1)
    diag = i2 == j2

    hard_tl = jnp.where(diag, 0.0, sim_tl)
    hard_tr = jnp.where(diag, 0.0, sim_tr)
    hard_bb = jnp.where(diag, 0.0, sim_bb)
    hard_bl = hard_tr.T

    m_top = jnp.maximum(jnp.max(hard_tl, axis=1, keepdims=True),
                        jnp.max(hard_tr, axis=1, keepdims=True))
    m_bot = jnp.maximum(jnp.max(hard_bl, axis=1, keepdims=True),
                        jnp.max(hard_bb, axis=1, keepdims=True))
    min_tl = jnp.min(jnp.where(hard_tl == m_top, j2, 512), axis=1, keepdims=True)
    min_tr = jnp.min(jnp.where(hard_tr == m_top, j2, 256), axis=1, keepdims=True)
    idx_top = jnp.minimum(min_tl, min_tr + 256)
    min_bl = jnp.min(jnp.where(hard_bl == m_bot, j2, 512), axis=1, keepdims=True)
    min_bb = jnp.min(jnp.where(hard_bb == m_bot, j2, 256), axis=1, keepdims=True)
    idx_bot = jnp.minimum(min_bl, min_bb + 256)

    mem_l = (jnp.any(idx_top == j2, axis=0, keepdims=True)
             | jnp.any(idx_bot == j2, axis=0, keepdims=True))
    mem_r = (jnp.any(idx_top - 256 == j2, axis=0, keepdims=True)
             | jnp.any(idx_bot - 256 == j2, axis=0, keepdims=True))

    pos_sel = ((j2 == i2 + 128) & (i2 < 128)) | ((j2 == i2 - 128) & (i2 >= 128))
    sum_pos = jnp.sum(jnp.where(pos_sel, sim_tl, 0.0), keepdims=True)
    contrastive = (jnp.sum(m_top, keepdims=True) + jnp.sum(m_bot, keepdims=True)
                   - 2.0 * sum_pos) / (_T * _H)

    sq_i = jnp.sum(zi * zi, axis=1, keepdims=True)
    sq_j = jnp.sum(zj * zj, axis=1, keepdims=True)
    s_i = jnp.sum(zi, axis=1, keepdims=True)
    s_j = jnp.sum(zj, axis=1, keepdims=True)
    a_term = sq_i + 2.0 * _EPS * s_i + (2.0 * _H * _EPS * _EPS)
    b_l = (sq_i - 2.0 * _EPS * s_i).T
    b_r = (sq_j - 2.0 * _EPS * s_j).T
    d2_l = a_term + b_l - 2.0 * sim_tl
    d2_r = a_term + b_r - 2.0 * sim_tr
    negd2 = jnp.minimum(
        jnp.min(jnp.where(mem_l, d2_l, _BIG), axis=1, keepdims=True),
        jnp.min(jnp.where(mem_r, d2_r, _BIG), axis=1, keepdims=True))
    neg_dist = jnp.sqrt(jnp.maximum(negd2, 0.0))

    dpos = zi - zj + _EPS
    pos_dist = jnp.sqrt(jnp.sum(dpos * dpos, axis=1, keepdims=True))

    triplet = jnp.sum(jnp.maximum(pos_dist - neg_dist + _MARGIN, 0.0),
                      keepdims=True) / _H
    out_ref[...] = triplet + _LAMBDA_C * contrastive


def kernel(img_emb, cap_emb, labels):
    out = pl.pallas_call(
        _loss_kernel,
        out_shape=jax.ShapeDtypeStruct((1, 1), jnp.float32),
    )(img_emb.astype(jnp.float32), cap_emb.astype(jnp.float32))
    return out[0, 0]
